# baseline (speedup 1.0000x reference)
import functools

import jax
import jax.numpy as jnp
from jax import lax
from jax.experimental import pallas as pl
from jax.experimental.pallas import tpu as pltpu
from jax.experimental.pallas import tpu_sc as plsc

H = 128
N_ATOMS = 10000
N_BONDS = 320000
MAX_NB = 32
N_MOLS = 500
NC, NS = 2, 16
NW = NC * NS
A_PAD = 10240
ATOMS_PER_W = A_PAD // NW
BONDS_PER_W = N_BONDS // NW
PRE_B = 80
PRE_ROUNDS = BONDS_PER_W // PRE_B


def _vsc_mesh():
    return plsc.VectorSubcoreMesh(
        core_axis_name="c", subcore_axis_name="s",
        num_cores=NC, num_subcores=NS)


def _mm0_body(fb_ref, wi_ref, wh_ref, inp_ref, mh_ref):
    x = jnp.dot(fb_ref[...], wi_ref[...], preferred_element_type=jnp.float32)
    inp_ref[...] = x
    mh_ref[...] = jnp.dot(jnp.maximum(x, 0.0), wh_ref[...],
                          preferred_element_type=jnp.float32)


def _mm0(f_bonds, W_i, W_h0, R=3200):
    nb, k = f_bonds.shape
    return pl.pallas_call(
        _mm0_body,
        grid=(nb // R,),
        in_specs=[pl.BlockSpec((R, k), lambda i: (i, 0)),
                  pl.BlockSpec((k, H), lambda i: (0, 0)),
                  pl.BlockSpec((H, H), lambda i: (0, 0))],
        out_specs=[pl.BlockSpec((R, H), lambda i: (i, 0)),
                   pl.BlockSpec((R, H), lambda i: (i, 0))],
        out_shape=[jax.ShapeDtypeStruct((nb, H), jnp.float32),
                   jax.ShapeDtypeStruct((nb, H), jnp.float32)],
    )(f_bonds, W_i, W_h0)


def _mm1_body(inp_ref, pre_ref, wh_ref, mh_ref):
    m = jnp.maximum(inp_ref[...] + pre_ref[...], 0.0)
    mh_ref[...] = jnp.dot(m, wh_ref[...], preferred_element_type=jnp.float32)


def _mm1(inp, pre, W_h, R=3200):
    nb = inp.shape[0]
    return pl.pallas_call(
        _mm1_body,
        grid=(nb // R,),
        in_specs=[pl.BlockSpec((R, H), lambda i: (i, 0)),
                  pl.BlockSpec((R, H), lambda i: (i, 0)),
                  pl.BlockSpec((H, H), lambda i: (0, 0))],
        out_specs=pl.BlockSpec((R, H), lambda i: (i, 0)),
        out_shape=jax.ShapeDtypeStruct((nb, H), jnp.float32),
    )(inp, pre, W_h)


def _ew_body(inp_ref, pre_ref, out_ref):
    out_ref[...] = jnp.maximum(inp_ref[...] + pre_ref[...], 0.0)


def _ew_relu_add(inp, pre, R=3200):
    nb = inp.shape[0]
    return pl.pallas_call(
        _ew_body,
        grid=(nb // R,),
        in_specs=[pl.BlockSpec((R, H), lambda i: (i, 0)),
                  pl.BlockSpec((R, H), lambda i: (i, 0))],
        out_specs=pl.BlockSpec((R, H), lambda i: (i, 0)),
        out_shape=jax.ShapeDtypeStruct((nb, H), jnp.float32),
    )(inp, pre)


def _readout_body(fa_ref, am_ref, woa_ref, wob_ref, bo_ref, sid_ref,
                  out_ref, acc_ref, cnt_ref):
    i = pl.program_id(0)

    @pl.when(i == 0)
    def _():
        acc_ref[...] = jnp.zeros_like(acc_ref)
        cnt_ref[...] = jnp.zeros_like(cnt_ref)

    h = jnp.maximum(
        jnp.dot(fa_ref[...], woa_ref[...], preferred_element_type=jnp.float32)
        + jnp.dot(am_ref[...], wob_ref[...], preferred_element_type=jnp.float32)
        + bo_ref[...], 0.0)
    ids = sid_ref[0, 0, :]
    iota = lax.broadcasted_iota(jnp.int32, (ids.shape[0], 512), 1)
    onehot = jnp.where(iota == ids[:, None], 1.0, 0.0)
    acc_ref[...] += lax.dot_general(onehot, h, (((0,), (0,)), ((), ())),
                                    preferred_element_type=jnp.float32)
    cnt_ref[...] += lax.dot_general(onehot, jnp.ones_like(h),
                                    (((0,), (0,)), ((), ())),
                                    preferred_element_type=jnp.float32)
    out_ref[...] = acc_ref[...] / jnp.maximum(cnt_ref[...], 1.0)


def _readout(f_atoms, am, W_oa, W_ob, b_o2, sid3, R=1000):
    na = f_atoms.shape[0]
    return pl.pallas_call(
        _readout_body,
        grid=(na // R,),
        in_specs=[pl.BlockSpec((R, H), lambda i: (i, 0)),
                  pl.BlockSpec((R, H), lambda i: (i, 0)),
                  pl.BlockSpec((H, H), lambda i: (0, 0)),
                  pl.BlockSpec((H, H), lambda i: (0, 0)),
                  pl.BlockSpec((1, H), lambda i: (0, 0)),
                  pl.BlockSpec((1, 1, R), lambda i: (i, 0, 0))],
        out_specs=pl.BlockSpec((512, H), lambda i: (0, 0)),
        out_shape=jax.ShapeDtypeStruct((512, H), jnp.float32),
        scratch_shapes=[pltpu.VMEM((512, H), jnp.float32),
                        pltpu.VMEM((512, H), jnp.float32)],
    )(f_atoms, am, W_oa, W_ob, b_o2, sid3)


def _scsum_body(mh_hbm, a2bf_hbm, amh_hbm, idx_v, rows_v, out_v, sem):
    w = lax.axis_index("s") * NC + lax.axis_index("c")
    base_atom = w * ATOMS_PER_W

    @pl.loop(0, ATOMS_PER_W // 4)
    def _round(i):
        a0 = base_atom + i * 4
        pltpu.sync_copy(a2bf_hbm.at[pl.ds(a0 * MAX_NB, 4 * MAX_NB)], idx_v)
        pltpu.async_copy(mh_hbm.at[idx_v], rows_v, sem).wait()
        for j in range(4):
            for c8 in range(8):
                acc = rows_v[j * MAX_NB, pl.ds(c8 * 16, 16)]
                for r in range(1, MAX_NB):
                    acc = acc + rows_v[j * MAX_NB + r, pl.ds(c8 * 16, 16)]
                out_v[j, pl.ds(c8 * 16, 16)] = acc
        pltpu.sync_copy(out_v, amh_hbm.at[pl.ds(a0, 4)])


def _scsum(mh, a2bf):
    f = pl.kernel(
        _scsum_body,
        out_type=jax.ShapeDtypeStruct((A_PAD, H), jnp.float32),
        mesh=_vsc_mesh(),
        scratch_types=[pltpu.VMEM((4 * MAX_NB,), jnp.int32),
                       pltpu.VMEM((4 * MAX_NB, H), jnp.float32),
                       pltpu.VMEM((4, H), jnp.float32),
                       pltpu.SemaphoreType.DMA],
    )
    return f(mh, a2bf)


def _scpre_body(amh_hbm, mh_hbm, b2a_hbm, b2revb_hbm, pre_hbm,
                idx1_v, idx2_v, rows1_v, rows2_v, sem1, sem2):
    w = lax.axis_index("s") * NC + lax.axis_index("c")
    base = w * BONDS_PER_W

    @pl.loop(0, PRE_ROUNDS)
    def _round(i):
        b0 = base + i * PRE_B
        pltpu.sync_copy(b2a_hbm.at[pl.ds(b0, PRE_B)], idx1_v)
        pltpu.sync_copy(b2revb_hbm.at[pl.ds(b0, PRE_B)], idx2_v)
        d1 = pltpu.async_copy(amh_hbm.at[idx1_v], rows1_v, sem1)
        d2 = pltpu.async_copy(mh_hbm.at[idx2_v], rows2_v, sem2)
        d1.wait()
        d2.wait()

        @pl.loop(0, PRE_B)
        def _row(r):
            for c8 in range(8):
                sl = pl.ds(c8 * 16, 16)
                rows1_v[r, sl] = rows1_v[r, sl] - rows2_v[r, sl]

        pltpu.sync_copy(rows1_v, pre_hbm.at[pl.ds(b0, PRE_B)])


def _scpre(amh, mh, b2a, b2revb):
    f = pl.kernel(
        _scpre_body,
        out_type=jax.ShapeDtypeStruct((N_BONDS, H), jnp.float32),
        mesh=_vsc_mesh(),
        scratch_types=[pltpu.VMEM((PRE_B,), jnp.int32),
                       pltpu.VMEM((PRE_B,), jnp.int32),
                       pltpu.VMEM((PRE_B, H), jnp.float32),
                       pltpu.VMEM((PRE_B, H), jnp.float32),
                       pltpu.SemaphoreType.DMA,
                       pltpu.SemaphoreType.DMA],
    )
    return f(amh, mh, b2a, b2revb)


def kernel(f_atoms, f_bonds, W_i, W_h0, W_h1, W_o, b_o, a2b, b2a, b2revb,
           scope_ids):
    inp, mh0 = _mm0(f_bonds, W_i, W_h0)
    a2bf = jnp.concatenate(
        [a2b.reshape(-1),
         jnp.zeros(((A_PAD - N_ATOMS) * MAX_NB,), jnp.int32)])
    amh0 = _scsum(mh0, a2bf)
    pre1 = _scpre(amh0, mh0, b2a, b2revb)
    mh1 = _mm1(inp, pre1, W_h1)
    amh1 = _scsum(mh1, a2bf)
    pre2 = _scpre(amh1, mh1, b2a, b2revb)
    m2 = _ew_relu_add(inp, pre2)
    am = _scsum(m2, a2bf)
    mol = _readout(f_atoms, am[:N_ATOMS], W_o[:H], W_o[H:],
                   b_o.reshape(1, H), scope_ids.reshape(10, 1, 1000))
    return mol[:N_MOLS]


# speedup vs baseline: 1.4013x; 1.4013x over previous
import functools

import jax
import jax.numpy as jnp
from jax import lax
from jax.experimental import pallas as pl
from jax.experimental.pallas import tpu as pltpu
from jax.experimental.pallas import tpu_sc as plsc

H = 128
N_ATOMS = 10000
N_BONDS = 320000
MAX_NB = 32
N_MOLS = 500
NC, NS = 2, 16
NW = NC * NS
A_PAD = 10240
ATOMS_PER_W = A_PAD // NW
BONDS_PER_W = N_BONDS // NW
PRE_B = 80
PRE_ROUNDS = BONDS_PER_W // PRE_B


def _vsc_mesh():
    return plsc.VectorSubcoreMesh(
        core_axis_name="c", subcore_axis_name="s",
        num_cores=NC, num_subcores=NS)


def _mm0_body(fb_ref, wi_ref, wh_ref, inp_ref, mh_ref):
    x = jnp.dot(fb_ref[...], wi_ref[...], preferred_element_type=jnp.float32)
    inp_ref[...] = x
    mh_ref[...] = jnp.dot(jnp.maximum(x, 0.0), wh_ref[...],
                          preferred_element_type=jnp.float32)


def _mm0(f_bonds, W_i, W_h0, R=3200):
    nb, k = f_bonds.shape
    return pl.pallas_call(
        _mm0_body,
        grid=(nb // R,),
        in_specs=[pl.BlockSpec((R, k), lambda i: (i, 0)),
                  pl.BlockSpec((k, H), lambda i: (0, 0)),
                  pl.BlockSpec((H, H), lambda i: (0, 0))],
        out_specs=[pl.BlockSpec((R, H), lambda i: (i, 0)),
                   pl.BlockSpec((R, H), lambda i: (i, 0))],
        out_shape=[jax.ShapeDtypeStruct((nb, H), jnp.float32),
                   jax.ShapeDtypeStruct((nb, H), jnp.float32)],
    )(f_bonds, W_i, W_h0)


def _mm1_body(inp_ref, pre_ref, wh_ref, mh_ref):
    m = jnp.maximum(inp_ref[...] + pre_ref[...], 0.0)
    mh_ref[...] = jnp.dot(m, wh_ref[...], preferred_element_type=jnp.float32)


def _mm1(inp, pre, W_h, R=3200):
    nb = inp.shape[0]
    return pl.pallas_call(
        _mm1_body,
        grid=(nb // R,),
        in_specs=[pl.BlockSpec((R, H), lambda i: (i, 0)),
                  pl.BlockSpec((R, H), lambda i: (i, 0)),
                  pl.BlockSpec((H, H), lambda i: (0, 0))],
        out_specs=pl.BlockSpec((R, H), lambda i: (i, 0)),
        out_shape=jax.ShapeDtypeStruct((nb, H), jnp.float32),
    )(inp, pre, W_h)


def _ew_body(inp_ref, pre_ref, out_ref):
    out_ref[...] = jnp.maximum(inp_ref[...] + pre_ref[...], 0.0)


def _ew_relu_add(inp, pre, R=3200):
    nb = inp.shape[0]
    return pl.pallas_call(
        _ew_body,
        grid=(nb // R,),
        in_specs=[pl.BlockSpec((R, H), lambda i: (i, 0)),
                  pl.BlockSpec((R, H), lambda i: (i, 0))],
        out_specs=pl.BlockSpec((R, H), lambda i: (i, 0)),
        out_shape=jax.ShapeDtypeStruct((nb, H), jnp.float32),
    )(inp, pre)


def _readout_body(fa_ref, am_ref, woa_ref, wob_ref, bo_ref, sid_ref,
                  out_ref, acc_ref, cnt_ref):
    i = pl.program_id(0)

    @pl.when(i == 0)
    def _():
        acc_ref[...] = jnp.zeros_like(acc_ref)
        cnt_ref[...] = jnp.zeros_like(cnt_ref)

    h = jnp.maximum(
        jnp.dot(fa_ref[...], woa_ref[...], preferred_element_type=jnp.float32)
        + jnp.dot(am_ref[...], wob_ref[...], preferred_element_type=jnp.float32)
        + bo_ref[...], 0.0)
    ids = sid_ref[0, 0, :]
    iota = lax.broadcasted_iota(jnp.int32, (ids.shape[0], 512), 1)
    onehot = jnp.where(iota == ids[:, None], 1.0, 0.0)
    acc_ref[...] += lax.dot_general(onehot, h, (((0,), (0,)), ((), ())),
                                    preferred_element_type=jnp.float32)
    cnt_ref[...] += lax.dot_general(onehot, jnp.ones_like(h),
                                    (((0,), (0,)), ((), ())),
                                    preferred_element_type=jnp.float32)
    out_ref[...] = acc_ref[...] / jnp.maximum(cnt_ref[...], 1.0)


def _readout(f_atoms, am, W_oa, W_ob, b_o2, sid3, R=1000):
    na = f_atoms.shape[0]
    return pl.pallas_call(
        _readout_body,
        grid=(na // R,),
        in_specs=[pl.BlockSpec((R, H), lambda i: (i, 0)),
                  pl.BlockSpec((R, H), lambda i: (i, 0)),
                  pl.BlockSpec((H, H), lambda i: (0, 0)),
                  pl.BlockSpec((H, H), lambda i: (0, 0)),
                  pl.BlockSpec((1, H), lambda i: (0, 0)),
                  pl.BlockSpec((1, 1, R), lambda i: (i, 0, 0))],
        out_specs=pl.BlockSpec((512, H), lambda i: (0, 0)),
        out_shape=jax.ShapeDtypeStruct((512, H), jnp.float32),
        scratch_shapes=[pltpu.VMEM((512, H), jnp.float32),
                        pltpu.VMEM((512, H), jnp.float32)],
    )(f_atoms, am, W_oa, W_ob, b_o2, sid3)


SS_NBUF = 4
SS_ROUNDS = ATOMS_PER_W // 4


def _scsum_body(mh_hbm, a2bf_hbm, amh_hbm, idx_all, rows0, rows1, rows2,
                rows3, out_all, sem0, sem1, sem2, sem3):
    w = lax.axis_index("s") * NC + lax.axis_index("c")
    base_atom = w * ATOMS_PER_W
    rows = (rows0, rows1, rows2, rows3)
    sems = (sem0, sem1, sem2, sem3)

    pltpu.sync_copy(
        a2bf_hbm.at[pl.ds(base_atom * MAX_NB, ATOMS_PER_W * MAX_NB)],
        idx_all)

    def _issue(r, b):
        pltpu.async_copy(mh_hbm.at[idx_all.at[pl.ds(r * 128, 128)]],
                         rows[b], sems[b])

    for b in range(SS_NBUF):
        _issue(b, b)

    @pl.loop(0, SS_ROUNDS // SS_NBUF)
    def _g(g):
        for b in range(SS_NBUF):
            r = g * SS_NBUF + b
            pltpu.make_async_copy(
                mh_hbm.at[idx_all.at[pl.ds(0, 128)]], rows[b],
                sems[b]).wait()
            @pl.loop(0, 4)
            def _atom(j):
                for c8 in range(8):
                    sl = pl.ds(c8 * 16, 16)
                    acc = rows[b][j * MAX_NB, sl]
                    for rr in range(1, MAX_NB):
                        acc = acc + rows[b][j * MAX_NB + rr, sl]
                    out_all[r * 4 + j, sl] = acc

            @pl.when(r + SS_NBUF < SS_ROUNDS)
            def _():
                _issue(r + SS_NBUF, b)

    pltpu.sync_copy(out_all, amh_hbm.at[pl.ds(base_atom, ATOMS_PER_W)])


def _scsum(mh, a2bf):
    f = pl.kernel(
        _scsum_body,
        out_type=jax.ShapeDtypeStruct((A_PAD, H), jnp.float32),
        mesh=_vsc_mesh(),
        scratch_types=[pltpu.VMEM((ATOMS_PER_W * MAX_NB,), jnp.int32)]
        + [pltpu.VMEM((128, H), jnp.float32)] * SS_NBUF
        + [pltpu.VMEM((ATOMS_PER_W, H), jnp.float32)]
        + [pltpu.SemaphoreType.DMA] * SS_NBUF,
    )
    return f(mh, a2bf)


PRE_CH = 128
PRE_FULL = BONDS_PER_W // PRE_CH
PRE_TAIL = BONDS_PER_W - PRE_FULL * PRE_CH


def _scpre_body(amh_hbm, mh_hbm, b2a_hbm, b2revb_hbm, pre_hbm,
                idx1_all, idx2_all, r1a, r1b, r2a, r2b, oa, ob,
                gsa, gsb, osa, osb):
    w = lax.axis_index("s") * NC + lax.axis_index("c")
    base = w * BONDS_PER_W
    rows1 = (r1a, r1b)
    rows2 = (r2a, r2b)
    outs = (oa, ob)
    gsems = (gsa, gsb)
    osems = (osa, osb)

    pltpu.sync_copy(b2a_hbm.at[pl.ds(base, BONDS_PER_W)], idx1_all)
    pltpu.sync_copy(b2revb_hbm.at[pl.ds(base, BONDS_PER_W)], idx2_all)

    def _issue(r, b):
        pltpu.async_copy(
            amh_hbm.at[idx1_all.at[pl.ds(r * PRE_CH, PRE_CH)]],
            rows1[b], gsems[b])
        pltpu.async_copy(
            mh_hbm.at[idx2_all.at[pl.ds(r * PRE_CH, PRE_CH)]],
            rows2[b], gsems[b])

    _issue(0, 0)
    _issue(1, 1)

    @pl.loop(0, PRE_FULL // 2)
    def _g(g):
        for b in range(2):
            r = g * 2 + b
            pltpu.make_async_copy(
                amh_hbm.at[idx1_all.at[pl.ds(0, PRE_CH)]], rows1[b],
                gsems[b]).wait()
            pltpu.make_async_copy(
                mh_hbm.at[idx2_all.at[pl.ds(0, PRE_CH)]], rows2[b],
                gsems[b]).wait()

            @pl.when(r >= 2)
            def _():
                pltpu.make_async_copy(
                    outs[b], pre_hbm.at[pl.ds(base, PRE_CH)],
                    osems[b]).wait()

            @pl.loop(0, PRE_CH)
            def _row(rr):
                for c8 in range(8):
                    sl = pl.ds(c8 * 16, 16)
                    outs[b][rr, sl] = rows1[b][rr, sl] - rows2[b][rr, sl]

            pltpu.async_copy(outs[b],
                             pre_hbm.at[pl.ds(base + r * PRE_CH, PRE_CH)],
                             osems[b])

            @pl.when(r + 2 < PRE_FULL)
            def _():
                _issue(r + 2, b)

    for b in range(2):
        pltpu.make_async_copy(outs[b], pre_hbm.at[pl.ds(base, PRE_CH)],
                              osems[b]).wait()

    toff = PRE_FULL * PRE_CH
    d1 = pltpu.async_copy(
        amh_hbm.at[idx1_all.at[pl.ds(toff, PRE_TAIL)]],
        r1a.at[pl.ds(0, PRE_TAIL)], gsa)
    d2 = pltpu.async_copy(
        mh_hbm.at[idx2_all.at[pl.ds(toff, PRE_TAIL)]],
        r2a.at[pl.ds(0, PRE_TAIL)], gsb)
    d1.wait()
    d2.wait()

    @pl.loop(0, PRE_TAIL)
    def _trow(rr):
        for c8 in range(8):
            sl = pl.ds(c8 * 16, 16)
            oa[rr, sl] = r1a[rr, sl] - r2a[rr, sl]

    pltpu.sync_copy(oa.at[pl.ds(0, PRE_TAIL)],
                    pre_hbm.at[pl.ds(base + toff, PRE_TAIL)])


def _scpre(amh, mh, b2a, b2revb):
    f = pl.kernel(
        _scpre_body,
        out_type=jax.ShapeDtypeStruct((N_BONDS, H), jnp.float32),
        mesh=_vsc_mesh(),
        scratch_types=[pltpu.VMEM((BONDS_PER_W,), jnp.int32)] * 2
        + [pltpu.VMEM((PRE_CH, H), jnp.float32)] * 6
        + [pltpu.SemaphoreType.DMA] * 4,
    )
    return f(amh, mh, b2a, b2revb)


def kernel(f_atoms, f_bonds, W_i, W_h0, W_h1, W_o, b_o, a2b, b2a, b2revb,
           scope_ids):
    inp, mh0 = _mm0(f_bonds, W_i, W_h0)
    a2bf = jnp.concatenate(
        [a2b.reshape(-1),
         jnp.zeros(((A_PAD - N_ATOMS) * MAX_NB,), jnp.int32)])
    amh0 = _scsum(mh0, a2bf)
    pre1 = _scpre(amh0, mh0, b2a, b2revb)
    mh1 = _mm1(inp, pre1, W_h1)
    amh1 = _scsum(mh1, a2bf)
    pre2 = _scpre(amh1, mh1, b2a, b2revb)
    m2 = _ew_relu_add(inp, pre2)
    am = _scsum(m2, a2bf)
    mol = _readout(f_atoms, am[:N_ATOMS], W_o[:H], W_o[H:],
                   b_o.reshape(1, H), scope_ids.reshape(10, 1, 1000))
    return mol[:N_MOLS]


# speedup vs baseline: 1.4115x; 1.0073x over previous
import functools

import jax
import jax.numpy as jnp
from jax import lax
from jax.experimental import pallas as pl
from jax.experimental.pallas import tpu as pltpu
from jax.experimental.pallas import tpu_sc as plsc

H = 128
N_ATOMS = 10000
N_BONDS = 320000
MAX_NB = 32
N_MOLS = 500
NC, NS = 2, 16
NW = NC * NS
A_PAD = 10240
ATOMS_PER_W = A_PAD // NW
BONDS_PER_W = N_BONDS // NW
PRE_B = 80
PRE_ROUNDS = BONDS_PER_W // PRE_B


def _vsc_mesh():
    return plsc.VectorSubcoreMesh(
        core_axis_name="c", subcore_axis_name="s",
        num_cores=NC, num_subcores=NS)


def _mm0_body(fb_ref, wi_ref, wh_ref, inp_ref, mh_ref):
    x = jnp.dot(fb_ref[...], wi_ref[...], preferred_element_type=jnp.float32)
    inp_ref[...] = x
    mh_ref[...] = jnp.dot(jnp.maximum(x, 0.0), wh_ref[...],
                          preferred_element_type=jnp.float32)


def _mm0(f_bonds, W_i, W_h0, R=3200):
    nb, k = f_bonds.shape
    return pl.pallas_call(
        _mm0_body,
        grid=(nb // R,),
        in_specs=[pl.BlockSpec((R, k), lambda i: (i, 0)),
                  pl.BlockSpec((k, H), lambda i: (0, 0)),
                  pl.BlockSpec((H, H), lambda i: (0, 0))],
        out_specs=[pl.BlockSpec((R, H), lambda i: (i, 0)),
                   pl.BlockSpec((R, H), lambda i: (i, 0))],
        out_shape=[jax.ShapeDtypeStruct((nb, H), jnp.float32),
                   jax.ShapeDtypeStruct((nb, H), jnp.float32)],
    )(f_bonds, W_i, W_h0)


def _mm1_body(inp_ref, pre_ref, wh_ref, mh_ref):
    m = jnp.maximum(inp_ref[...] + pre_ref[...], 0.0)
    mh_ref[...] = jnp.dot(m, wh_ref[...], preferred_element_type=jnp.float32)


def _mm1(inp, pre, W_h, R=3200):
    nb = inp.shape[0]
    return pl.pallas_call(
        _mm1_body,
        grid=(nb // R,),
        in_specs=[pl.BlockSpec((R, H), lambda i: (i, 0)),
                  pl.BlockSpec((R, H), lambda i: (i, 0)),
                  pl.BlockSpec((H, H), lambda i: (0, 0))],
        out_specs=pl.BlockSpec((R, H), lambda i: (i, 0)),
        out_shape=jax.ShapeDtypeStruct((nb, H), jnp.float32),
    )(inp, pre, W_h)


def _ew_body(inp_ref, pre_ref, out_ref):
    out_ref[...] = jnp.maximum(inp_ref[...] + pre_ref[...], 0.0)


def _ew_relu_add(inp, pre, R=3200):
    nb = inp.shape[0]
    return pl.pallas_call(
        _ew_body,
        grid=(nb // R,),
        in_specs=[pl.BlockSpec((R, H), lambda i: (i, 0)),
                  pl.BlockSpec((R, H), lambda i: (i, 0))],
        out_specs=pl.BlockSpec((R, H), lambda i: (i, 0)),
        out_shape=jax.ShapeDtypeStruct((nb, H), jnp.float32),
    )(inp, pre)


def _readout_body(fa_ref, am_ref, woa_ref, wob_ref, bo_ref, sid_ref,
                  out_ref, acc_ref, cnt_ref):
    i = pl.program_id(0)

    @pl.when(i == 0)
    def _():
        acc_ref[...] = jnp.zeros_like(acc_ref)
        cnt_ref[...] = jnp.zeros_like(cnt_ref)

    h = jnp.maximum(
        jnp.dot(fa_ref[...], woa_ref[...], preferred_element_type=jnp.float32)
        + jnp.dot(am_ref[...], wob_ref[...], preferred_element_type=jnp.float32)
        + bo_ref[...], 0.0)
    ids = sid_ref[0, 0, :]
    iota = lax.broadcasted_iota(jnp.int32, (ids.shape[0], 512), 1)
    onehot = jnp.where(iota == ids[:, None], 1.0, 0.0)
    acc_ref[...] += lax.dot_general(onehot, h, (((0,), (0,)), ((), ())),
                                    preferred_element_type=jnp.float32)
    cnt_ref[...] += lax.dot_general(onehot, jnp.ones_like(h),
                                    (((0,), (0,)), ((), ())),
                                    preferred_element_type=jnp.float32)
    out_ref[...] = acc_ref[...] / jnp.maximum(cnt_ref[...], 1.0)


def _readout(f_atoms, am, W_oa, W_ob, b_o2, sid3, R=1000):
    na = f_atoms.shape[0]
    return pl.pallas_call(
        _readout_body,
        grid=(na // R,),
        in_specs=[pl.BlockSpec((R, H), lambda i: (i, 0)),
                  pl.BlockSpec((R, H), lambda i: (i, 0)),
                  pl.BlockSpec((H, H), lambda i: (0, 0)),
                  pl.BlockSpec((H, H), lambda i: (0, 0)),
                  pl.BlockSpec((1, H), lambda i: (0, 0)),
                  pl.BlockSpec((1, 1, R), lambda i: (i, 0, 0))],
        out_specs=pl.BlockSpec((512, H), lambda i: (0, 0)),
        out_shape=jax.ShapeDtypeStruct((512, H), jnp.float32),
        scratch_shapes=[pltpu.VMEM((512, H), jnp.float32),
                        pltpu.VMEM((512, H), jnp.float32)],
    )(f_atoms, am, W_oa, W_ob, b_o2, sid3)


SS_NBUF = 4
SS_ROUNDS = ATOMS_PER_W // 4


def _scsum_body(mh_hbm, a2bf_hbm, amh_hbm, idx_all, rows0, rows1, rows2,
                rows3, out_all, sem0, sem1, sem2, sem3):
    w = lax.axis_index("s") * NC + lax.axis_index("c")
    base_atom = w * ATOMS_PER_W
    rows = (rows0, rows1, rows2, rows3)
    sems = (sem0, sem1, sem2, sem3)

    pltpu.sync_copy(
        a2bf_hbm.at[pl.ds(base_atom * MAX_NB, ATOMS_PER_W * MAX_NB)],
        idx_all)

    def _issue(r, b):
        pltpu.async_copy(mh_hbm.at[idx_all.at[pl.ds(r * 128, 128)]],
                         rows[b], sems[b])

    for b in range(SS_NBUF):
        _issue(b, b)

    @pl.loop(0, SS_ROUNDS // SS_NBUF)
    def _g(g):
        for b in range(SS_NBUF):
            r = g * SS_NBUF + b
            pltpu.make_async_copy(
                mh_hbm.at[idx_all.at[pl.ds(0, 128)]], rows[b],
                sems[b]).wait()
            @pl.loop(0, 4)
            def _atom(j):
                accs = [rows[b][j * MAX_NB, pl.ds(c8 * 16, 16)]
                        for c8 in range(8)]
                for rr in range(1, MAX_NB):
                    for c8 in range(8):
                        accs[c8] = accs[c8] + rows[b][j * MAX_NB + rr,
                                                      pl.ds(c8 * 16, 16)]
                for c8 in range(8):
                    out_all[r * 4 + j, pl.ds(c8 * 16, 16)] = accs[c8]

            @pl.when(r + SS_NBUF < SS_ROUNDS)
            def _():
                _issue(r + SS_NBUF, b)

    pltpu.sync_copy(out_all, amh_hbm.at[pl.ds(base_atom, ATOMS_PER_W)])


def _scsum(mh, a2bf):
    f = pl.kernel(
        _scsum_body,
        out_type=jax.ShapeDtypeStruct((A_PAD, H), jnp.float32),
        mesh=_vsc_mesh(),
        scratch_types=[pltpu.VMEM((ATOMS_PER_W * MAX_NB,), jnp.int32)]
        + [pltpu.VMEM((128, H), jnp.float32)] * SS_NBUF
        + [pltpu.VMEM((ATOMS_PER_W, H), jnp.float32)]
        + [pltpu.SemaphoreType.DMA] * SS_NBUF,
    )
    return f(mh, a2bf)


PRE_CH = 128
PRE_FULL = BONDS_PER_W // PRE_CH
PRE_TAIL = BONDS_PER_W - PRE_FULL * PRE_CH


def _scpre_body(amh_hbm, mh_hbm, b2a_hbm, b2revb_hbm, pre_hbm,
                idx1_all, idx2_all, r1a, r1b, r2a, r2b, oa, ob,
                gsa, gsb, osa, osb):
    w = lax.axis_index("s") * NC + lax.axis_index("c")
    base = w * BONDS_PER_W
    rows1 = (r1a, r1b)
    rows2 = (r2a, r2b)
    outs = (oa, ob)
    gsems = (gsa, gsb)
    osems = (osa, osb)

    pltpu.sync_copy(b2a_hbm.at[pl.ds(base, BONDS_PER_W)], idx1_all)
    pltpu.sync_copy(b2revb_hbm.at[pl.ds(base, BONDS_PER_W)], idx2_all)

    def _issue(r, b):
        pltpu.async_copy(
            amh_hbm.at[idx1_all.at[pl.ds(r * PRE_CH, PRE_CH)]],
            rows1[b], gsems[b])
        pltpu.async_copy(
            mh_hbm.at[idx2_all.at[pl.ds(r * PRE_CH, PRE_CH)]],
            rows2[b], gsems[b])

    _issue(0, 0)
    _issue(1, 1)

    @pl.loop(0, PRE_FULL // 2)
    def _g(g):
        for b in range(2):
            r = g * 2 + b
            pltpu.make_async_copy(
                amh_hbm.at[idx1_all.at[pl.ds(0, PRE_CH)]], rows1[b],
                gsems[b]).wait()
            pltpu.make_async_copy(
                mh_hbm.at[idx2_all.at[pl.ds(0, PRE_CH)]], rows2[b],
                gsems[b]).wait()

            @pl.when(r >= 2)
            def _():
                pltpu.make_async_copy(
                    outs[b], pre_hbm.at[pl.ds(base, PRE_CH)],
                    osems[b]).wait()

            @pl.loop(0, PRE_CH)
            def _row(rr):
                for c8 in range(8):
                    sl = pl.ds(c8 * 16, 16)
                    outs[b][rr, sl] = rows1[b][rr, sl] - rows2[b][rr, sl]

            pltpu.async_copy(outs[b],
                             pre_hbm.at[pl.ds(base + r * PRE_CH, PRE_CH)],
                             osems[b])

            @pl.when(r + 2 < PRE_FULL)
            def _():
                _issue(r + 2, b)

    for b in range(2):
        pltpu.make_async_copy(outs[b], pre_hbm.at[pl.ds(base, PRE_CH)],
                              osems[b]).wait()

    toff = PRE_FULL * PRE_CH
    d1 = pltpu.async_copy(
        amh_hbm.at[idx1_all.at[pl.ds(toff, PRE_TAIL)]],
        r1a.at[pl.ds(0, PRE_TAIL)], gsa)
    d2 = pltpu.async_copy(
        mh_hbm.at[idx2_all.at[pl.ds(toff, PRE_TAIL)]],
        r2a.at[pl.ds(0, PRE_TAIL)], gsb)
    d1.wait()
    d2.wait()

    @pl.loop(0, PRE_TAIL)
    def _trow(rr):
        for c8 in range(8):
            sl = pl.ds(c8 * 16, 16)
            oa[rr, sl] = r1a[rr, sl] - r2a[rr, sl]

    pltpu.sync_copy(oa.at[pl.ds(0, PRE_TAIL)],
                    pre_hbm.at[pl.ds(base + toff, PRE_TAIL)])


def _scpre(amh, mh, b2a, b2revb):
    f = pl.kernel(
        _scpre_body,
        out_type=jax.ShapeDtypeStruct((N_BONDS, H), jnp.float32),
        mesh=_vsc_mesh(),
        scratch_types=[pltpu.VMEM((BONDS_PER_W,), jnp.int32)] * 2
        + [pltpu.VMEM((PRE_CH, H), jnp.float32)] * 6
        + [pltpu.SemaphoreType.DMA] * 4,
    )
    return f(amh, mh, b2a, b2revb)


def kernel(f_atoms, f_bonds, W_i, W_h0, W_h1, W_o, b_o, a2b, b2a, b2revb,
           scope_ids):
    inp, mh0 = _mm0(f_bonds, W_i, W_h0)
    a2bf = jnp.concatenate(
        [a2b.reshape(-1),
         jnp.zeros(((A_PAD - N_ATOMS) * MAX_NB,), jnp.int32)])
    amh0 = _scsum(mh0, a2bf)
    pre1 = _scpre(amh0, mh0, b2a, b2revb)
    mh1 = _mm1(inp, pre1, W_h1)
    amh1 = _scsum(mh1, a2bf)
    pre2 = _scpre(amh1, mh1, b2a, b2revb)
    m2 = _ew_relu_add(inp, pre2)
    am = _scsum(m2, a2bf)
    mol = _readout(f_atoms, am[:N_ATOMS], W_o[:H], W_o[H:],
                   b_o.reshape(1, H), scope_ids.reshape(10, 1, 1000))
    return mol[:N_MOLS]


# speedup vs baseline: 2.3569x; 1.6698x over previous
import functools

import jax
import jax.numpy as jnp
from jax import lax
from jax.experimental import pallas as pl
from jax.experimental.pallas import tpu as pltpu
from jax.experimental.pallas import tpu_sc as plsc

H = 128
N_ATOMS = 10000
N_BONDS = 320000
MAX_NB = 32
N_MOLS = 500
NC, NS = 2, 16
NW = NC * NS
A_PAD = 10240
ATOMS_PER_W = A_PAD // NW
BONDS_PER_W = N_BONDS // NW
PRE_B = 80
PRE_ROUNDS = BONDS_PER_W // PRE_B


def _vsc_mesh():
    return plsc.VectorSubcoreMesh(
        core_axis_name="c", subcore_axis_name="s",
        num_cores=NC, num_subcores=NS)


def _mm0_body(fb_ref, wi_ref, wh_ref, inp_ref, mh_ref):
    x = jnp.dot(fb_ref[...], wi_ref[...], preferred_element_type=jnp.float32)
    inp_ref[...] = x
    m = jnp.maximum(x, 0.0)
    mh_ref[...] = jnp.dot(m, wh_ref[...], preferred_element_type=jnp.float32)


def _mm0(f_bonds, W_i, W_h, R=3200):
    nb, k = f_bonds.shape
    return pl.pallas_call(
        _mm0_body,
        grid=(nb // R,),
        in_specs=[pl.BlockSpec((R, k), lambda i: (i, 0)),
                  pl.BlockSpec((k, H), lambda i: (0, 0)),
                  pl.BlockSpec((H, H), lambda i: (0, 0))],
        out_specs=[pl.BlockSpec((R, H), lambda i: (i, 0)),
                   pl.BlockSpec((R, H), lambda i: (i, 0))],
        out_shape=[jax.ShapeDtypeStruct((nb, H), jnp.float32),
                   jax.ShapeDtypeStruct((nb, H), jnp.float32)],
    )(f_bonds, W_i, W_h)


def _mm1_body(inp_ref, pre_ref, wh_ref, mh_ref):
    m = jnp.maximum(inp_ref[...] + pre_ref[...].astype(jnp.float32), 0.0)
    mh_ref[...] = jnp.dot(m, wh_ref[...], preferred_element_type=jnp.float32)


def _mm1(inp, pre, W_h, R=3200):
    nb = inp.shape[0]
    return pl.pallas_call(
        _mm1_body,
        grid=(nb // R,),
        in_specs=[pl.BlockSpec((R, H), lambda i: (i, 0)),
                  pl.BlockSpec((R, H), lambda i: (i, 0)),
                  pl.BlockSpec((H, H), lambda i: (0, 0))],
        out_specs=pl.BlockSpec((R, H), lambda i: (i, 0)),
        out_shape=jax.ShapeDtypeStruct((nb, H), jnp.float32),
    )(inp, pre, W_h)


def _ew_body(inp_ref, pre_ref, out_ref):
    out_ref[...] = jnp.maximum(
        inp_ref[...] + pre_ref[...].astype(jnp.float32), 0.0)


def _ew_relu_add(inp, pre, R=3200):
    nb = inp.shape[0]
    return pl.pallas_call(
        _ew_body,
        grid=(nb // R,),
        in_specs=[pl.BlockSpec((R, H), lambda i: (i, 0)),
                  pl.BlockSpec((R, H), lambda i: (i, 0))],
        out_specs=pl.BlockSpec((R, H), lambda i: (i, 0)),
        out_shape=jax.ShapeDtypeStruct((nb, H), jnp.float32),
    )(inp, pre)


def _readout_body(fa_ref, am_ref, woa_ref, wob_ref, bo_ref, sid_ref,
                  out_ref, acc_ref, cnt_ref):
    i = pl.program_id(0)

    @pl.when(i == 0)
    def _():
        acc_ref[...] = jnp.zeros_like(acc_ref)
        cnt_ref[...] = jnp.zeros_like(cnt_ref)

    h = jnp.maximum(
        jnp.dot(fa_ref[...], woa_ref[...], preferred_element_type=jnp.float32)
        + jnp.dot(am_ref[...], wob_ref[...], preferred_element_type=jnp.float32)
        + bo_ref[...], 0.0)
    ids = sid_ref[0, 0, :]
    iota = lax.broadcasted_iota(jnp.int32, (ids.shape[0], 512), 1)
    onehot = jnp.where(iota == ids[:, None], 1.0, 0.0)
    acc_ref[...] += lax.dot_general(onehot, h, (((0,), (0,)), ((), ())),
                                    preferred_element_type=jnp.float32)
    cnt_ref[...] += lax.dot_general(onehot, jnp.ones_like(h),
                                    (((0,), (0,)), ((), ())),
                                    preferred_element_type=jnp.float32)
    out_ref[...] = acc_ref[...] / jnp.maximum(cnt_ref[...], 1.0)


def _readout(f_atoms, am, W_oa, W_ob, b_o2, sid3, R=1000):
    na = f_atoms.shape[0]
    return pl.pallas_call(
        _readout_body,
        grid=(na // R,),
        in_specs=[pl.BlockSpec((R, H), lambda i: (i, 0)),
                  pl.BlockSpec((R, H), lambda i: (i, 0)),
                  pl.BlockSpec((H, H), lambda i: (0, 0)),
                  pl.BlockSpec((H, H), lambda i: (0, 0)),
                  pl.BlockSpec((1, H), lambda i: (0, 0)),
                  pl.BlockSpec((1, 1, R), lambda i: (i, 0, 0))],
        out_specs=pl.BlockSpec((512, H), lambda i: (0, 0)),
        out_shape=jax.ShapeDtypeStruct((512, H), jnp.float32),
        scratch_shapes=[pltpu.VMEM((512, H), jnp.float32),
                        pltpu.VMEM((512, H), jnp.float32)],
    )(f_atoms, am, W_oa, W_ob, b_o2, sid3)


SS_NBUF = 4
SS_ROUNDS = ATOMS_PER_W // 4


def _scsum_body(out_bf16, mh_hbm, a2bf_hbm, amh_hbm, idx_all, rows0, rows1,
                rows2, rows3, out_all, sem0, sem1, sem2, sem3):
    w = lax.axis_index("s") * NC + lax.axis_index("c")
    base_atom = w * ATOMS_PER_W
    rows = (rows0, rows1, rows2, rows3)
    sems = (sem0, sem1, sem2, sem3)

    pltpu.sync_copy(
        a2bf_hbm.at[pl.ds(base_atom * MAX_NB, ATOMS_PER_W * MAX_NB)],
        idx_all)

    def _issue(r, b):
        pltpu.async_copy(mh_hbm.at[idx_all.at[pl.ds(r * 128, 128)]],
                         rows[b], sems[b])

    for b in range(SS_NBUF):
        _issue(b, b)

    @pl.loop(0, SS_ROUNDS // SS_NBUF)
    def _g(g):
        for b in range(SS_NBUF):
            r = g * SS_NBUF + b
            pltpu.make_async_copy(
                mh_hbm.at[idx_all.at[pl.ds(0, 128)]], rows[b],
                sems[b]).wait()
            @pl.loop(0, 4)
            def _atom(j):
                accs = [rows[b][j * MAX_NB, pl.ds(c8 * 16, 16)]
                        for c8 in range(8)]
                for rr in range(1, MAX_NB):
                    for c8 in range(8):
                        accs[c8] = accs[c8] + rows[b][j * MAX_NB + rr,
                                                      pl.ds(c8 * 16, 16)]
                if out_bf16:
                    for k in range(4):
                        out_all[r * 4 + j, pl.ds(k * 32, 32)] = plsc.pack(
                            accs[2 * k], accs[2 * k + 1],
                            format=plsc.PackFormat.INTERLEAVED)
                else:
                    for c8 in range(8):
                        out_all[r * 4 + j, pl.ds(c8 * 16, 16)] = accs[c8]

            @pl.when(r + SS_NBUF < SS_ROUNDS)
            def _():
                _issue(r + SS_NBUF, b)

    pltpu.sync_copy(out_all, amh_hbm.at[pl.ds(base_atom, ATOMS_PER_W)])


def _scsum(mh, a2bf, out_bf16=False):
    odt = jnp.bfloat16 if out_bf16 else jnp.float32
    f = pl.kernel(
        functools.partial(_scsum_body, out_bf16),
        out_type=jax.ShapeDtypeStruct((A_PAD, H), odt),
        mesh=_vsc_mesh(),
        scratch_types=[pltpu.VMEM((ATOMS_PER_W * MAX_NB,), jnp.int32)]
        + [pltpu.VMEM((128, H), jnp.float32)] * SS_NBUF
        + [pltpu.VMEM((ATOMS_PER_W, H), odt)]
        + [pltpu.SemaphoreType.DMA] * SS_NBUF,
    )
    return f(mh, a2bf)


PRE_CH = 80
PRE_FULL = BONDS_PER_W // PRE_CH
PRE_NB = 5


def _scpre_body(amh_hbm, mh_hbm, b2a_hbm, b2revb_hbm, pre_hbm,
                idx1_all, idx2_all,
                r10, r11, r12, r13, r14, r20, r21, r22, r23, r24,
                gs0, gs1, gs2, gs3, gs4, os0, os1, os2, os3, os4):
    w = lax.axis_index("s") * NC + lax.axis_index("c")
    base = w * BONDS_PER_W
    rows1 = (r10, r11, r12, r13, r14)
    rows2 = (r20, r21, r22, r23, r24)
    gsems = (gs0, gs1, gs2, gs3, gs4)
    osems = (os0, os1, os2, os3, os4)

    pltpu.sync_copy(b2a_hbm.at[pl.ds(base, BONDS_PER_W)], idx1_all)
    pltpu.sync_copy(b2revb_hbm.at[pl.ds(base, BONDS_PER_W)], idx2_all)

    def _issue(r, b):
        pltpu.async_copy(
            amh_hbm.at[idx1_all.at[pl.ds(r * PRE_CH, PRE_CH)]],
            rows1[b], gsems[b])
        pltpu.async_copy(
            mh_hbm.at[idx2_all.at[pl.ds(r * PRE_CH, PRE_CH)]],
            rows2[b], gsems[b])

    for b in range(PRE_NB - 1):
        _issue(b, b)

    @pl.loop(0, PRE_FULL // PRE_NB)
    def _g(g):
        for b in range(PRE_NB):
            r = g * PRE_NB + b
            pltpu.make_async_copy(
                amh_hbm.at[idx1_all.at[pl.ds(0, PRE_CH)]], rows1[b],
                gsems[b]).wait()
            pltpu.make_async_copy(
                mh_hbm.at[idx2_all.at[pl.ds(0, PRE_CH)]], rows2[b],
                gsems[b]).wait()

            @pl.loop(0, PRE_CH)
            def _row(rr):
                for c8 in range(8):
                    sl = pl.ds(c8 * 16, 16)
                    rows1[b][rr, sl] = rows1[b][rr, sl] - rows2[b][rr, sl]

            pltpu.async_copy(rows1[b],
                             pre_hbm.at[pl.ds(base + r * PRE_CH, PRE_CH)],
                             osems[b])

            @pl.when(r + PRE_NB - 1 < PRE_FULL)
            def _():
                nxt = (b + PRE_NB - 1) % PRE_NB

                @pl.when(r >= 1)
                def _():
                    pltpu.make_async_copy(
                        rows1[nxt], pre_hbm.at[pl.ds(base, PRE_CH)],
                        osems[nxt]).wait()
                _issue(r + PRE_NB - 1, nxt)

    for b in range(PRE_NB):
        pltpu.make_async_copy(rows1[b], pre_hbm.at[pl.ds(base, PRE_CH)],
                              osems[b]).wait()


def _scpre(amh, mh, b2a, b2revb):
    f = pl.kernel(
        _scpre_body,
        out_type=jax.ShapeDtypeStruct((N_BONDS, H), jnp.float32),
        mesh=_vsc_mesh(),
        scratch_types=[pltpu.VMEM((BONDS_PER_W,), jnp.int32)] * 2
        + [pltpu.VMEM((PRE_CH, H), jnp.float32)] * (2 * PRE_NB)
        + [pltpu.SemaphoreType.DMA] * (2 * PRE_NB),
    )
    return f(amh, mh, b2a, b2revb)


def kernel(f_atoms, f_bonds, W_i, W_h0, W_h1, W_o, b_o, a2b, b2a, b2revb,
           scope_ids):
    inp, mh0 = _mm0(f_bonds, W_i, W_h0)
    pad = jnp.arange((A_PAD - N_ATOMS) * MAX_NB, dtype=jnp.int32) % N_BONDS
    a2bf = jnp.concatenate([a2b.reshape(-1), pad])
    amh0 = _scsum(mh0, a2bf)
    pre1 = _scpre(amh0, mh0, b2a, b2revb)
    mh1 = _mm1(inp, pre1, W_h1)
    amh1 = _scsum(mh1, a2bf)
    pre2 = _scpre(amh1, mh1, b2a, b2revb)
    m2 = _ew_relu_add(inp, pre2)
    am = _scsum(m2, a2bf)
    mol = _readout(f_atoms, am[:N_ATOMS], W_o[:H], W_o[H:],
                   b_o.reshape(1, H), scope_ids.reshape(10, 1, 1000))
    return mol[:N_MOLS]
